# baseline (speedup 1.0000x reference)
import functools

import jax
import jax.numpy as jnp
from jax import lax
from jax.experimental import pallas as pl
from jax.experimental.pallas import tpu as pltpu
from jax.experimental.pallas import tpu_sc as plsc

_BS = 8
_DIM = 768
_LMAX = 96
_MARGIN = 1.0
_EPS2 = 1e-16
_NCHUNK = _DIM // 16
_JQ = _LMAX // 4


def _rsqrt_newton(x):
    i = lax.bitcast_convert_type(x, jnp.int32)
    i = jnp.int32(0x5F3759DF) - (i >> 1)
    y = lax.bitcast_convert_type(i, jnp.float32)
    for _ in range(3):
        y = y * (1.5 - 0.5 * x * y * y)
    return y


def _sc_body(anchor_hbm, pos_hbm, neg_hbm, plen_hbm, nlen_hbm,
             out_hbm,
             plen_v, nlen_v, off_v, idxp_v, idxn_v, anchor_v,
             rows_v, dist_v, dn_v, red_v, dots2d_v, nn2d_v,
             shared, seml, sema, semp, semn):
    c = lax.axis_index("c")
    s = lax.axis_index("s")
    seg = c * 4 + s // 4
    seg_local = s // 4
    q = s % 4
    lanes16 = jnp.arange(16, dtype=jnp.int32)
    seg_idx = jnp.full((16,), seg, jnp.int32)
    lane15 = jnp.full((16,), 15, jnp.int32)
    zeros16 = jnp.zeros((16,), jnp.int32)

    cp_a = pltpu.async_copy(anchor_hbm, anchor_v, sema)

    plen_v[...] = zeros16
    nlen_v[...] = zeros16
    cp_l0 = pltpu.async_copy(plen_hbm, plen_v.at[pl.ds(0, _BS)], seml)
    cp_l1 = pltpu.async_copy(nlen_hbm, nlen_v.at[pl.ds(0, _BS)], seml)
    cp_l0.wait()
    cp_l1.wait()
    plens = plen_v[...]
    nlens = nlen_v[...]
    off_v[0] = plsc.cumsum(plens) - plens
    off_v[1] = plsc.cumsum(nlens) - nlens
    p_offb = plsc.load_gather(off_v, [zeros16, seg_idx])
    n_offb = plsc.load_gather(off_v, [zeros16 + 1, seg_idx])

    base = q * _JQ
    idxp_v[pl.ds(0, 16)] = p_offb + base + lanes16
    idxp_v[pl.ds(8, 16)] = p_offb + base + 8 + lanes16
    idxn_v[pl.ds(0, 16)] = n_offb + base + lanes16
    idxn_v[pl.ds(8, 16)] = n_offb + base + 8 + lanes16

    cp_p = pltpu.async_copy(pos_hbm.at[idxp_v], rows_v.at[pl.ds(0, _JQ)], semp)
    cp_n = pltpu.async_copy(neg_hbm.at[idxn_v], rows_v.at[pl.ds(_JQ, _JQ)], semn)

    cp_a.wait()
    def na_body(cg, na):
        for cc in range(4):
            av = anchor_v[seg, pl.ds((cg * 4 + cc) * 16, 16)]
            na = na + av * av
        return na

    na_acc = lax.fori_loop(0, _NCHUNK // 4, na_body,
                           jnp.zeros((16,), jnp.float32))
    dots2d_v[0] = plsc.cumsum(na_acc)
    na2b = plsc.load_gather(dots2d_v, [zeros16, lane15])
    inv_na = _rsqrt_newton(jnp.maximum(na2b, _EPS2))

    def row_body(rp, carry):
        r0 = rp * 4
        def chunk_body(cg, carry):
            d0, d1, d2, d3, n0, n1, n2, n3 = carry
            d = [d0, d1, d2, d3]
            n = [n0, n1, n2, n3]
            for cc in range(4):
                av = anchor_v[seg, pl.ds((cg * 4 + cc) * 16, 16)]
                for u in range(4):
                    xv = rows_v[r0 + u, pl.ds((cg * 4 + cc) * 16, 16)]
                    d[u] = d[u] + av * xv
                    n[u] = n[u] + xv * xv
            return tuple(d) + tuple(n)

        z = jnp.zeros((16,), jnp.float32)
        cres = lax.fori_loop(0, _NCHUNK // 4, chunk_body, (z,) * 8)
        d = list(cres[:4])
        n = list(cres[4:])
        for u in range(4):
            dots2d_v[r0 + u] = plsc.cumsum(d[u])
            nn2d_v[r0 + u] = plsc.cumsum(n[u])
        return carry

    cp_p.wait()
    lax.fori_loop(0, _JQ // 4, row_body, 0)
    cp_n.wait()
    lax.fori_loop(_JQ // 4, _JQ // 2, row_body, 0)
    for g in range(3):
        lo = lanes16 + g * 16
        dotv = plsc.load_gather(dots2d_v, [lo, lane15])
        nnv = plsc.load_gather(nn2d_v, [lo, lane15])
        inv_nx = _rsqrt_newton(jnp.maximum(nnv, _EPS2))
        dist_v[pl.ds(g * 16, 16)] = 1.0 - dotv * inv_nx * inv_na

    cp_d0 = pltpu.async_copy(dist_v.at[pl.ds(0, _JQ)],
                             shared.at[seg_local, 0, q, pl.ds(0, _JQ)], seml)
    cp_d1 = pltpu.async_copy(dist_v.at[pl.ds(_JQ, _JQ)],
                             shared.at[seg_local, 1, q, pl.ds(0, _JQ)], seml)
    cp_d0.wait()
    cp_d1.wait()
    plsc.subcore_barrier()

    cp_f = [pltpu.async_copy(shared.at[seg_local, 1, t, pl.ds(0, _JQ)],
                             dn_v.at[pl.ds(t * _JQ, _JQ)], seml)
            for t in range(4)]
    for cp in cp_f:
        cp.wait()

    plb = plsc.load_gather(plen_v, [seg_idx])
    nlb = plsc.load_gather(nlen_v, [seg_idx])
    zero16f = jnp.zeros((16,), jnp.float32)

    def hinge_j(j, acc):
        dpj = plsc.load_gather(dist_v, [jnp.full((16,), j, jnp.int32)])
        jmask = jnp.full((16,), base + j, jnp.int32) < plb
        for kc in range(_LMAX // 16):
            dnk = dn_v[pl.ds(kc * 16, 16)]
            kmask = (lanes16 + kc * 16) < nlb
            h = jnp.maximum(dpj - dnk + _MARGIN, 0.0)
            acc = acc + jnp.where(jmask & kmask, h, zero16f)
        return acc

    acc = lax.fori_loop(0, _JQ, hinge_j, jnp.zeros((16,), jnp.float32))

    wid = c * 16 + s
    red_v[...] = acc
    pltpu.sync_copy(red_v, out_hbm.at[pl.ds(wid * 16, 16)])

    @pl.when((s == 0) & (c == 0))
    def _():
        red_v[...] = (plens * nlens).astype(jnp.float32)
        pltpu.sync_copy(red_v, out_hbm.at[pl.ds(512, 16)])


@jax.jit
def _wsdm_sc(anchor, positive, negative, plens, nlens):
    mesh = plsc.VectorSubcoreMesh(core_axis_name="c", subcore_axis_name="s")
    kern = functools.partial(
        pl.kernel,
        out_type=jax.ShapeDtypeStruct((528,), jnp.float32),
        mesh=mesh,
        compiler_params=pltpu.CompilerParams(needs_layout_passes=False),
        scratch_types=[
            pltpu.VMEM((16,), jnp.int32),
            pltpu.VMEM((16,), jnp.int32),
            pltpu.VMEM((2, 16), jnp.int32),
            pltpu.VMEM((_JQ,), jnp.int32),
            pltpu.VMEM((_JQ,), jnp.int32),
            pltpu.VMEM((_BS, _DIM), jnp.float32),
            pltpu.VMEM((2 * _JQ, _DIM), jnp.float32),
            pltpu.VMEM((2 * _JQ,), jnp.float32),
            pltpu.VMEM((_LMAX,), jnp.float32),
            pltpu.VMEM((16,), jnp.float32),
            pltpu.VMEM((2 * _JQ, 16), jnp.float32),
            pltpu.VMEM((2 * _JQ, 16), jnp.float32),
            pltpu.VMEM_SHARED((4, 2, 4, 128), jnp.float32),
            pltpu.SemaphoreType.DMA,
            pltpu.SemaphoreType.DMA,
            pltpu.SemaphoreType.DMA,
            pltpu.SemaphoreType.DMA,
        ],
    )(_sc_body)
    return kern(anchor, positive, negative, plens, nlens)


def kernel(anchor, positive, negative, positive_lens, negative_lens):
    out = _wsdm_sc(anchor, positive, negative,
                   positive_lens.astype(jnp.int32),
                   negative_lens.astype(jnp.int32))
    sums = jnp.sum(out.reshape(33, 16), axis=1)
    return jnp.sum(sums[:32]) / sums[32]


# speedup vs baseline: 1.0029x; 1.0029x over previous
import functools

import jax
import jax.numpy as jnp
from jax import lax
from jax.experimental import pallas as pl
from jax.experimental.pallas import tpu as pltpu
from jax.experimental.pallas import tpu_sc as plsc

_BS = 8
_DIM = 768
_LMAX = 96
_MARGIN = 1.0
_EPS2 = 1e-16
_NCHUNK = _DIM // 16
_JQ = _LMAX // 4


def _rsqrt_newton(x):
    i = lax.bitcast_convert_type(x, jnp.int32)
    i = jnp.int32(0x5F3759DF) - (i >> 1)
    y = lax.bitcast_convert_type(i, jnp.float32)
    for _ in range(3):
        y = y * (1.5 - 0.5 * x * y * y)
    return y


def _sc_body(anchor_hbm, pos_hbm, neg_hbm, plen_hbm, nlen_hbm,
             out_hbm,
             plen_v, nlen_v, off_v, idxp_v, idxn_v, anchor_v,
             rows_v, dist_v, dn_v, red_v, dots2d_v, nn2d_v,
             shared, seml, sema, semp, semn):
    c = lax.axis_index("c")
    s = lax.axis_index("s")
    seg = c * 4 + s // 4
    seg_local = s // 4
    q = s % 4
    lanes16 = jnp.arange(16, dtype=jnp.int32)
    seg_idx = jnp.full((16,), seg, jnp.int32)
    lane15 = jnp.full((16,), 15, jnp.int32)
    zeros16 = jnp.zeros((16,), jnp.int32)

    cp_a = pltpu.async_copy(anchor_hbm, anchor_v, sema)

    plen_v[...] = zeros16
    nlen_v[...] = zeros16
    cp_l0 = pltpu.async_copy(plen_hbm, plen_v.at[pl.ds(0, _BS)], seml)
    cp_l1 = pltpu.async_copy(nlen_hbm, nlen_v.at[pl.ds(0, _BS)], seml)
    cp_l0.wait()
    cp_l1.wait()
    plens = plen_v[...]
    nlens = nlen_v[...]
    off_v[0] = plsc.cumsum(plens) - plens
    off_v[1] = plsc.cumsum(nlens) - nlens
    p_offb = plsc.load_gather(off_v, [zeros16, seg_idx])
    n_offb = plsc.load_gather(off_v, [zeros16 + 1, seg_idx])

    base = q * _JQ
    idxp_v[pl.ds(0, 16)] = p_offb + base + lanes16
    idxp_v[pl.ds(8, 16)] = p_offb + base + 8 + lanes16
    idxn_v[pl.ds(0, 16)] = n_offb + base + lanes16
    idxn_v[pl.ds(8, 16)] = n_offb + base + 8 + lanes16

    cp_p = pltpu.async_copy(pos_hbm.at[idxp_v], rows_v.at[pl.ds(0, _JQ)], semp)
    cp_n = pltpu.async_copy(neg_hbm.at[idxn_v], rows_v.at[pl.ds(_JQ, _JQ)], semn)

    cp_a.wait()
    def na_body(cg, na):
        for cc in range(4):
            av = anchor_v[seg, pl.ds((cg * 4 + cc) * 16, 16)]
            na = na + av * av
        return na

    na_acc = lax.fori_loop(0, _NCHUNK // 4, na_body,
                           jnp.zeros((16,), jnp.float32))
    dots2d_v[0] = plsc.cumsum(na_acc)
    na2b = plsc.load_gather(dots2d_v, [zeros16, lane15])
    inv_na = _rsqrt_newton(jnp.maximum(na2b, _EPS2))

    def row_body(rp, carry):
        r0 = rp * 4
        def chunk_body(cg, carry):
            d0, d1, d2, d3, n0, n1, n2, n3 = carry
            d = [d0, d1, d2, d3]
            n = [n0, n1, n2, n3]
            for cc in range(4):
                av = anchor_v[seg, pl.ds((cg * 4 + cc) * 16, 16)]
                for u in range(4):
                    xv = rows_v[r0 + u, pl.ds((cg * 4 + cc) * 16, 16)]
                    d[u] = d[u] + av * xv
                    n[u] = n[u] + xv * xv
            return tuple(d) + tuple(n)

        z = jnp.zeros((16,), jnp.float32)
        cres = lax.fori_loop(0, _NCHUNK // 4, chunk_body, (z,) * 8)
        d = list(cres[:4])
        n = list(cres[4:])
        for u in range(4):
            dots2d_v[r0 + u] = plsc.cumsum(d[u])
            nn2d_v[r0 + u] = plsc.cumsum(n[u])
        return carry

    cp_p.wait()
    cp_n.wait()
    lax.fori_loop(0, _JQ // 2, row_body, 0)
    for g in range(3):
        lo = lanes16 + g * 16
        dotv = plsc.load_gather(dots2d_v, [lo, lane15])
        nnv = plsc.load_gather(nn2d_v, [lo, lane15])
        inv_nx = _rsqrt_newton(jnp.maximum(nnv, _EPS2))
        dist_v[pl.ds(g * 16, 16)] = 1.0 - dotv * inv_nx * inv_na

    cp_d0 = pltpu.async_copy(dist_v.at[pl.ds(0, _JQ)],
                             shared.at[seg_local, 0, q, pl.ds(0, _JQ)], seml)
    cp_d1 = pltpu.async_copy(dist_v.at[pl.ds(_JQ, _JQ)],
                             shared.at[seg_local, 1, q, pl.ds(0, _JQ)], seml)
    cp_d0.wait()
    cp_d1.wait()
    plsc.subcore_barrier()

    cp_f = [pltpu.async_copy(shared.at[seg_local, 1, t, pl.ds(0, _JQ)],
                             dn_v.at[pl.ds(t * _JQ, _JQ)], seml)
            for t in range(4)]
    for cp in cp_f:
        cp.wait()

    plb = plsc.load_gather(plen_v, [seg_idx])
    nlb = plsc.load_gather(nlen_v, [seg_idx])
    zero16f = jnp.zeros((16,), jnp.float32)

    def hinge_j(j, acc):
        dpj = plsc.load_gather(dist_v, [jnp.full((16,), j, jnp.int32)])
        jmask = jnp.full((16,), base + j, jnp.int32) < plb
        for kc in range(_LMAX // 16):
            dnk = dn_v[pl.ds(kc * 16, 16)]
            kmask = (lanes16 + kc * 16) < nlb
            h = jnp.maximum(dpj - dnk + _MARGIN, 0.0)
            acc = acc + jnp.where(jmask & kmask, h, zero16f)
        return acc

    acc = lax.fori_loop(0, _JQ, hinge_j, jnp.zeros((16,), jnp.float32))

    wid = c * 16 + s
    red_v[...] = acc
    pltpu.sync_copy(red_v, out_hbm.at[pl.ds(wid * 16, 16)])

    @pl.when((s == 0) & (c == 0))
    def _():
        red_v[...] = (plens * nlens).astype(jnp.float32)
        pltpu.sync_copy(red_v, out_hbm.at[pl.ds(512, 16)])


@jax.jit
def _wsdm_sc(anchor, positive, negative, plens, nlens):
    mesh = plsc.VectorSubcoreMesh(core_axis_name="c", subcore_axis_name="s")
    kern = functools.partial(
        pl.kernel,
        out_type=jax.ShapeDtypeStruct((528,), jnp.float32),
        mesh=mesh,
        compiler_params=pltpu.CompilerParams(needs_layout_passes=False),
        scratch_types=[
            pltpu.VMEM((16,), jnp.int32),
            pltpu.VMEM((16,), jnp.int32),
            pltpu.VMEM((2, 16), jnp.int32),
            pltpu.VMEM((_JQ,), jnp.int32),
            pltpu.VMEM((_JQ,), jnp.int32),
            pltpu.VMEM((_BS, _DIM), jnp.float32),
            pltpu.VMEM((2 * _JQ, _DIM), jnp.float32),
            pltpu.VMEM((2 * _JQ,), jnp.float32),
            pltpu.VMEM((_LMAX,), jnp.float32),
            pltpu.VMEM((16,), jnp.float32),
            pltpu.VMEM((2 * _JQ, 16), jnp.float32),
            pltpu.VMEM((2 * _JQ, 16), jnp.float32),
            pltpu.VMEM_SHARED((4, 2, 4, 128), jnp.float32),
            pltpu.SemaphoreType.DMA,
            pltpu.SemaphoreType.DMA,
            pltpu.SemaphoreType.DMA,
            pltpu.SemaphoreType.DMA,
        ],
    )(_sc_body)
    return kern(anchor, positive, negative, plens, nlens)


def kernel(anchor, positive, negative, positive_lens, negative_lens):
    out = _wsdm_sc(anchor, positive, negative,
                   positive_lens.astype(jnp.int32),
                   negative_lens.astype(jnp.int32))
    sums = jnp.sum(out.reshape(33, 16), axis=1)
    return jnp.sum(sums[:32]) / sums[32]


# speedup vs baseline: 1.0053x; 1.0024x over previous
import functools

import jax
import jax.numpy as jnp
from jax import lax
from jax.experimental import pallas as pl
from jax.experimental.pallas import tpu as pltpu
from jax.experimental.pallas import tpu_sc as plsc

_BS = 8
_DIM = 768
_LMAX = 96
_MARGIN = 1.0
_EPS2 = 1e-16
_NCHUNK = _DIM // 16
_JQ = _LMAX // 4


def _rsqrt_newton(x):
    i = lax.bitcast_convert_type(x, jnp.int32)
    i = jnp.int32(0x5F3759DF) - (i >> 1)
    y = lax.bitcast_convert_type(i, jnp.float32)
    for _ in range(3):
        y = y * (1.5 - 0.5 * x * y * y)
    return y


def _sc_body(anchor_hbm, pos_hbm, neg_hbm, plen_hbm, nlen_hbm,
             out_hbm,
             plen_v, nlen_v, off_v, idxp_v, idxn_v, anchor_v,
             rows_v, dist_v, dn_v, red_v, dots2d_v, nn2d_v,
             shared, seml, sema, semp, semn):
    c = lax.axis_index("c")
    s = lax.axis_index("s")
    seg = c * 4 + s // 4
    seg_local = s // 4
    q = s % 4
    lanes16 = jnp.arange(16, dtype=jnp.int32)
    seg_idx = jnp.full((16,), seg, jnp.int32)
    lane15 = jnp.full((16,), 15, jnp.int32)
    zeros16 = jnp.zeros((16,), jnp.int32)

    cp_a = pltpu.async_copy(anchor_hbm, anchor_v, sema)

    plen_v[...] = zeros16
    nlen_v[...] = zeros16
    cp_l0 = pltpu.async_copy(plen_hbm, plen_v.at[pl.ds(0, _BS)], seml)
    cp_l1 = pltpu.async_copy(nlen_hbm, nlen_v.at[pl.ds(0, _BS)], seml)
    cp_l0.wait()
    cp_l1.wait()
    plens = plen_v[...]
    nlens = nlen_v[...]
    off_v[0] = plsc.cumsum(plens) - plens
    off_v[1] = plsc.cumsum(nlens) - nlens
    p_offb = plsc.load_gather(off_v, [zeros16, seg_idx])
    n_offb = plsc.load_gather(off_v, [zeros16 + 1, seg_idx])

    base = q * _JQ
    idxp_v[pl.ds(0, 16)] = p_offb + base + lanes16
    idxp_v[pl.ds(8, 16)] = p_offb + base + 8 + lanes16
    idxn_v[pl.ds(0, 16)] = n_offb + base + lanes16
    idxn_v[pl.ds(8, 16)] = n_offb + base + 8 + lanes16

    cp_p = pltpu.async_copy(pos_hbm.at[idxp_v], rows_v.at[pl.ds(0, _JQ)], semp)
    cp_n = pltpu.async_copy(neg_hbm.at[idxn_v], rows_v.at[pl.ds(_JQ, _JQ)], semn)

    cp_a.wait()
    def na_body(cg, na):
        for cc in range(4):
            av = anchor_v[seg, pl.ds((cg * 4 + cc) * 16, 16)]
            na = na + av * av
        return na

    na_acc = lax.fori_loop(0, _NCHUNK // 4, na_body,
                           jnp.zeros((16,), jnp.float32))
    dots2d_v[0] = plsc.cumsum(na_acc)
    na2b = plsc.load_gather(dots2d_v, [zeros16, lane15])
    inv_na = _rsqrt_newton(jnp.maximum(na2b, _EPS2))

    def row_body(rp, carry):
        r0 = rp * 4
        def chunk_body(cg, carry):
            d0, d1, d2, d3, n0, n1, n2, n3 = carry
            d = [d0, d1, d2, d3]
            n = [n0, n1, n2, n3]
            for cc in range(4):
                av = anchor_v[seg, pl.ds((cg * 4 + cc) * 16, 16)]
                for u in range(4):
                    xv = rows_v[r0 + u, pl.ds((cg * 4 + cc) * 16, 16)]
                    d[u] = d[u] + av * xv
                    n[u] = n[u] + xv * xv
            return tuple(d) + tuple(n)

        z = jnp.zeros((16,), jnp.float32)
        cres = lax.fori_loop(0, _NCHUNK // 4, chunk_body, (z,) * 8)
        d = list(cres[:4])
        n = list(cres[4:])
        for u in range(4):
            dots2d_v[r0 + u] = plsc.cumsum(d[u])
            nn2d_v[r0 + u] = plsc.cumsum(n[u])
        return carry

    cp_p.wait()
    cp_n.wait()
    lax.fori_loop(0, _JQ // 2, row_body, 0)
    def dist_body(g, carry):
        lo = lanes16 + g * 16
        dotv = plsc.load_gather(dots2d_v, [lo, lane15])
        nnv = plsc.load_gather(nn2d_v, [lo, lane15])
        inv_nx = _rsqrt_newton(jnp.maximum(nnv, _EPS2))
        dist_v[pl.ds(g * 16, 16)] = 1.0 - dotv * inv_nx * inv_na
        return carry

    lax.fori_loop(0, 3, dist_body, 0)

    cp_d0 = pltpu.async_copy(dist_v.at[pl.ds(0, _JQ)],
                             shared.at[seg_local, 0, q, pl.ds(0, _JQ)], seml)
    cp_d1 = pltpu.async_copy(dist_v.at[pl.ds(_JQ, _JQ)],
                             shared.at[seg_local, 1, q, pl.ds(0, _JQ)], seml)
    cp_d0.wait()
    cp_d1.wait()
    plsc.subcore_barrier()

    cp_f = [pltpu.async_copy(shared.at[seg_local, 1, t, pl.ds(0, _JQ)],
                             dn_v.at[pl.ds(t * _JQ, _JQ)], seml)
            for t in range(4)]
    for cp in cp_f:
        cp.wait()

    plb = plsc.load_gather(plen_v, [seg_idx])
    nlb = plsc.load_gather(nlen_v, [seg_idx])
    zero16f = jnp.zeros((16,), jnp.float32)

    def hinge_j(j, acc):
        dpj = plsc.load_gather(dist_v, [jnp.full((16,), j, jnp.int32)])
        jmask = jnp.full((16,), base + j, jnp.int32) < plb

        def k_body(kc, a):
            dnk = dn_v[pl.ds(kc * 16, 16)]
            kmask = (lanes16 + kc * 16) < nlb
            h = jnp.maximum(dpj - dnk + _MARGIN, 0.0)
            return a + jnp.where(jmask & kmask, h, zero16f)

        return lax.fori_loop(0, _LMAX // 16, k_body, acc)

    acc = lax.fori_loop(0, _JQ, hinge_j, jnp.zeros((16,), jnp.float32))

    wid = c * 16 + s
    red_v[...] = acc
    pltpu.sync_copy(red_v, out_hbm.at[pl.ds(wid * 16, 16)])

    @pl.when((s == 0) & (c == 0))
    def _():
        red_v[...] = (plens * nlens).astype(jnp.float32)
        pltpu.sync_copy(red_v, out_hbm.at[pl.ds(512, 16)])


@jax.jit
def _wsdm_sc(anchor, positive, negative, plens, nlens):
    mesh = plsc.VectorSubcoreMesh(core_axis_name="c", subcore_axis_name="s")
    kern = functools.partial(
        pl.kernel,
        out_type=jax.ShapeDtypeStruct((528,), jnp.float32),
        mesh=mesh,
        compiler_params=pltpu.CompilerParams(needs_layout_passes=False),
        scratch_types=[
            pltpu.VMEM((16,), jnp.int32),
            pltpu.VMEM((16,), jnp.int32),
            pltpu.VMEM((2, 16), jnp.int32),
            pltpu.VMEM((_JQ,), jnp.int32),
            pltpu.VMEM((_JQ,), jnp.int32),
            pltpu.VMEM((_BS, _DIM), jnp.float32),
            pltpu.VMEM((2 * _JQ, _DIM), jnp.float32),
            pltpu.VMEM((2 * _JQ,), jnp.float32),
            pltpu.VMEM((_LMAX,), jnp.float32),
            pltpu.VMEM((16,), jnp.float32),
            pltpu.VMEM((2 * _JQ, 16), jnp.float32),
            pltpu.VMEM((2 * _JQ, 16), jnp.float32),
            pltpu.VMEM_SHARED((4, 2, 4, 128), jnp.float32),
            pltpu.SemaphoreType.DMA,
            pltpu.SemaphoreType.DMA,
            pltpu.SemaphoreType.DMA,
            pltpu.SemaphoreType.DMA,
        ],
    )(_sc_body)
    return kern(anchor, positive, negative, plens, nlens)


def kernel(anchor, positive, negative, positive_lens, negative_lens):
    out = _wsdm_sc(anchor, positive, negative,
                   positive_lens.astype(jnp.int32),
                   negative_lens.astype(jnp.int32))
    sums = jnp.sum(out.reshape(33, 16), axis=1)
    return jnp.sum(sums[:32]) / sums[32]
